# Initial kernel scaffold; baseline (speedup 1.0000x reference)
import jax
import jax.numpy as jnp
from jax.experimental import pallas as pl


def kernel(x, edge_index, edge_weight, W1, b1, W2, b2):
    raise NotImplementedError("write your pallas kernel here")


# speedup vs baseline: 8.2805x; 8.2805x over previous
import functools

import jax
import jax.numpy as jnp
from jax import lax
from jax.experimental import pallas as pl
from jax.experimental.pallas import tpu as pltpu
from jax.experimental.pallas import tpu_sc as plsc

NC = 2
NS = 16
NW = NC * NS
CH = 128


def _wid():
    return lax.axis_index("c") * NS + lax.axis_index("s")


def _make_deg_kernel(N, EPW):
    mesh = plsc.VectorSubcoreMesh(core_axis_name="c", subcore_axis_name="s")

    @functools.partial(
        pl.kernel,
        out_type=jax.ShapeDtypeStruct((NW, N), jnp.float32),
        mesh=mesh,
        scratch_types=[
            pltpu.VMEM((EPW,), jnp.int32),
            pltpu.VMEM((EPW,), jnp.float32),
            pltpu.VMEM((N,), jnp.float32),
        ],
        compiler_params=pltpu.CompilerParams(needs_layout_passes=False),
    )
    def deg_kernel(dst_hbm, ew_hbm, out_hbm, dstv, ewv, degl):
        wid = _wid()
        base = wid * EPW
        pltpu.sync_copy(dst_hbm.at[pl.ds(base, EPW)], dstv)
        pltpu.sync_copy(ew_hbm.at[pl.ds(base, EPW)], ewv)

        zero16 = jnp.zeros((16,), jnp.float32)

        @pl.loop(0, N // 16)
        def _(i):
            degl[pl.ds(i * 16, 16)] = zero16

        @pl.loop(0, EPW // 16)
        def _(g):
            d16 = dstv[pl.ds(g * 16, 16)]
            e16 = ewv[pl.ds(g * 16, 16)]
            plsc.addupdate_scatter(degl, [d16], e16)

        pltpu.sync_copy(degl, out_hbm.at[wid])

    return deg_kernel


def _make_agg_kernel(N, H, CPW, STR, LASTR):
    mesh = plsc.VectorSubcoreMesh(core_axis_name="c", subcore_axis_name="s")

    @functools.partial(
        pl.kernel,
        out_type=jax.ShapeDtypeStruct((NC, N, H), jnp.float32),
        mesh=mesh,
        scratch_types=[
            pltpu.VMEM((N,), jnp.float32),
            pltpu.VMEM((CPW, CH), jnp.int32),
            pltpu.VMEM((CPW, CH), jnp.int32),
            pltpu.VMEM((CPW, CH), jnp.float32),
            pltpu.VMEM((CH, H), jnp.float32),
            pltpu.VMEM((CH,), jnp.float32),
            pltpu.VMEM_SHARED((N, H), jnp.float32),
            pltpu.SemaphoreType.DMA,
        ],
        compiler_params=pltpu.CompilerParams(needs_layout_passes=False,
                                             use_tc_tiling_on_sc=False),
    )
    def agg_kernel(mat_hbm, src_hbm, dst_hbm, ew_hbm, dinv_hbm, zero_hbm,
                   out_hbm, dinvv, srcv, dstv, ewv, rows, normb, accS, sem):
        c = lax.axis_index("c")
        s = lax.axis_index("s")
        rbase = _wid() * CPW
        pltpu.sync_copy(dinv_hbm, dinvv)
        pltpu.sync_copy(src_hbm.at[pl.ds(rbase, CPW)], srcv)
        pltpu.sync_copy(dst_hbm.at[pl.ds(rbase, CPW)], dstv)
        pltpu.sync_copy(ew_hbm.at[pl.ds(rbase, CPW)], ewv)

        @pl.when(s < NS - 1)
        def _():
            pltpu.sync_copy(zero_hbm.at[pl.ds(s * STR, STR)],
                            accS.at[pl.ds(s * STR, STR)])

        @pl.when(s == NS - 1)
        def _():
            pltpu.sync_copy(zero_hbm.at[pl.ds((NS - 1) * STR, LASTR)],
                            accS.at[pl.ds((NS - 1) * STR, LASTR)])

        plsc.subcore_barrier()

        @pl.loop(0, CPW)
        def _(ch):
            pltpu.async_copy(mat_hbm.at[srcv.at[ch]], rows, sem).wait()
            for g in range(CH // 16):
                s16 = srcv[ch, pl.ds(g * 16, 16)]
                d16 = dstv[ch, pl.ds(g * 16, 16)]
                e16 = ewv[ch, pl.ds(g * 16, 16)]
                nv = (plsc.load_gather(dinvv, [s16]) * e16 *
                      plsc.load_gather(dinvv, [d16]))
                normb[pl.ds(g * 16, 16)] = nv
            for e in range(CH):
                nb = plsc.load_gather(normb, [jnp.full((16,), e, jnp.int32)])
                rows[e, pl.ds(0, 16)] = rows[e, pl.ds(0, 16)] * nb
                rows[e, pl.ds(16, 16)] = rows[e, pl.ds(16, 16)] * nb
            pltpu.sync_copy(rows, accS.at[dstv.at[ch]], add=True)

        plsc.subcore_barrier()

        @pl.when(s < NS - 1)
        def _():
            pltpu.sync_copy(accS.at[pl.ds(s * STR, STR)],
                            out_hbm.at[c, pl.ds(s * STR, STR)])

        @pl.when(s == NS - 1)
        def _():
            pltpu.sync_copy(accS.at[pl.ds((NS - 1) * STR, LASTR)],
                            out_hbm.at[c, pl.ds((NS - 1) * STR, LASTR)])

    return agg_kernel


def _xw(x, W1, BR):
    N, D = x.shape
    H = W1.shape[1]

    def body(x_ref, w_ref, xw_ref):
        xw_ref[...] = jnp.dot(x_ref[...], w_ref[...],
                              preferred_element_type=jnp.float32)

    return pl.pallas_call(
        body,
        grid=(N // BR,),
        in_specs=[
            pl.BlockSpec((BR, D), lambda i: (i, 0)),
            pl.BlockSpec((D, H), lambda i: (0, 0)),
        ],
        out_specs=pl.BlockSpec((BR, H), lambda i: (i, 0)),
        out_shape=jax.ShapeDtypeStruct((N, H), jnp.float32),
    )(x, W1)


def _dinv(degp):
    N = degp.shape[1]

    def body(degp_ref, dinv_ref):
        deg = jnp.sum(degp_ref[...], axis=0) + 1.0
        dinv_ref[...] = lax.rsqrt(deg)

    return pl.pallas_call(
        body,
        out_shape=jax.ShapeDtypeStruct((N,), jnp.float32),
    )(degp)


def _combine_relu(p, mat, dinv_col, b1_row, BR):
    N, H = mat.shape

    def body(p0_ref, p1_ref, m_ref, d_ref, b_ref, h_ref):
        d = d_ref[...]
        t = p0_ref[...] + p1_ref[...] + (d * d) * m_ref[...] + b_ref[...]
        h_ref[...] = jnp.maximum(t, 0.0)

    return pl.pallas_call(
        body,
        grid=(N // BR,),
        in_specs=[
            pl.BlockSpec((BR, H), lambda i: (i, 0)),
            pl.BlockSpec((BR, H), lambda i: (i, 0)),
            pl.BlockSpec((BR, H), lambda i: (i, 0)),
            pl.BlockSpec((BR, 1), lambda i: (i, 0)),
            pl.BlockSpec((1, H), lambda i: (0, 0)),
        ],
        out_specs=pl.BlockSpec((BR, H), lambda i: (i, 0)),
        out_shape=jax.ShapeDtypeStruct((N, H), jnp.float32),
    )(p[0], p[1], mat, dinv_col, b1_row)


def _combine_matvec(q, mat, dinv_col, w2_row, b2, BR):
    N, H = mat.shape

    def body(q0_ref, q1_ref, m_ref, d_ref, w_ref, b_ref, o_ref):
        d = d_ref[...]
        g = q0_ref[...] + q1_ref[...] + (d * d) * m_ref[...]
        o_ref[...] = jnp.sum(g * w_ref[...], axis=1, keepdims=True) + b_ref[...]

    return pl.pallas_call(
        body,
        grid=(N // BR,),
        in_specs=[
            pl.BlockSpec((BR, H), lambda i: (i, 0)),
            pl.BlockSpec((BR, H), lambda i: (i, 0)),
            pl.BlockSpec((BR, H), lambda i: (i, 0)),
            pl.BlockSpec((BR, 1), lambda i: (i, 0)),
            pl.BlockSpec((1, H), lambda i: (0, 0)),
            pl.BlockSpec((1, 1), lambda i: (0, 0)),
        ],
        out_specs=pl.BlockSpec((BR, 1), lambda i: (i, 0)),
        out_shape=jax.ShapeDtypeStruct((N, 1), jnp.float32),
    )(q[0], q[1], mat, dinv_col, w2_row, b2)


def kernel(x, edge_index, edge_weight, W1, b1, W2, b2):
    N, D = x.shape
    H = W1.shape[1]
    E = edge_index.shape[1]

    CPW = -(-E // (NW * CH))
    EPW = CPW * CH
    E_pad = EPW * NW
    STR = -(-N // NS) // 8 * 8 + 8
    LASTR = N - (NS - 1) * STR
    BR = 1000

    pad = E_pad - E
    src = jnp.pad(edge_index[0], (0, pad))
    dst = jnp.pad(edge_index[1], (0, pad))
    ew = jnp.pad(edge_weight, (0, pad))
    src2 = src.reshape(E_pad // CH, CH)
    dst2 = dst.reshape(E_pad // CH, CH)
    ew2 = ew.reshape(E_pad // CH, CH)
    zero32 = jnp.zeros((N, H), jnp.float32)

    degp = jnp.zeros((N,), jnp.float32).at[dst].add(ew)[None]
    xw = _xw(x, W1, BR)
    dinv = _dinv(degp)
    dinv_col = dinv[:, None]
    b1_row = b1.reshape(1, H)

    agg = _make_agg_kernel(N, H, CPW, STR, LASTR)
    p = agg(xw, src2, dst2, ew2, dinv, zero32)
    h = _combine_relu(p, xw, dinv_col, b1_row, BR)
    q = agg(h, src2, dst2, ew2, dinv, zero32)
    out = _combine_matvec(q, h, dinv_col, W2.reshape(1, H), b2.reshape(1, 1),
                          BR)
    return out
